# baseline (device time: 71868 ns/iter reference)
import jax
import jax.numpy as jnp
from jax import lax
from jax.experimental import pallas as pl
from jax.experimental.pallas import tpu as pltpu

N_DEV = 16

_printed = [False]


def _print_topology_once():
    if _printed[0]:
        return
    _printed[0] = True
    try:
        ds = jax.devices()
        print(f"[kernel] n_devices={len(ds)}")
        for d in ds[:32]:
            print(f"[kernel] id={d.id} coords={getattr(d, 'coords', None)} "
                  f"core={getattr(d, 'core_on_chip', None)}")
    except Exception as e:
        print(f"[kernel] topology probe failed: {e}")


def kernel(x, w_mat):
    _print_topology_once()
    m_total, k_shard = x.shape
    k_total, n = w_mat.shape
    m_blk = m_total // N_DEV

    x = x.astype(jnp.bfloat16)
    w_mat = w_mat.astype(jnp.bfloat16)

    def body(x_ref, w_ref, out_ref, xg_ref, amax_ref,
             send_sems, recv_sems, a_send_sems, a_recv_sems, local_sem):
        me = lax.axis_index("i")

        barrier_sem = pltpu.get_barrier_semaphore()
        for d in range(1, N_DEV):
            peer = (me + d) % N_DEV
            pl.semaphore_signal(barrier_sem, inc=1, device_id=(peer,),
                                device_id_type=pl.DeviceIdType.MESH)
        pl.semaphore_wait(barrier_sem, N_DEV - 1)

        sends = []
        for d in range(1, N_DEV):
            dest = (me + d) % N_DEV
            rdma = pltpu.make_async_remote_copy(
                src_ref=x_ref.at[pl.ds(dest * m_blk, m_blk), :],
                dst_ref=xg_ref.at[me],
                send_sem=send_sems.at[dest],
                recv_sem=recv_sems.at[me],
                device_id=(dest,),
                device_id_type=pl.DeviceIdType.MESH,
            )
            rdma.start()
            sends.append(rdma)

        cp = pltpu.make_async_copy(
            x_ref.at[pl.ds(me * m_blk, m_blk), :], xg_ref.at[me], local_sem)
        cp.start()
        cp.wait()

        def block_dot(j):
            a = xg_ref[j]
            b = w_ref[pl.ds(j * k_shard, k_shard), :]
            return lax.dot_general(a, b, (((1,), (0,)), ((), ())),
                                   preferred_element_type=jnp.float32)

        acc = block_dot(me)
        for d in range(1, N_DEV):
            src = (me - d + N_DEV) % N_DEV
            recv = pltpu.make_async_remote_copy(
                src_ref=xg_ref.at[src],
                dst_ref=xg_ref.at[src],
                send_sem=send_sems.at[src],
                recv_sem=recv_sems.at[src],
                device_id=(src,),
                device_id_type=pl.DeviceIdType.MESH,
            )
            recv.wait_recv()
            acc = acc + block_dot(src)

        local_amax = jnp.maximum(jnp.max(acc), 0.0)
        amax_ref[me] = jnp.full((8, 128), local_amax, jnp.float32)
        a_sends = []
        for d in range(1, N_DEV):
            dest = (me + d) % N_DEV
            rdma = pltpu.make_async_remote_copy(
                src_ref=amax_ref.at[me],
                dst_ref=amax_ref.at[me],
                send_sem=a_send_sems.at[dest],
                recv_sem=a_recv_sems.at[me],
                device_id=(dest,),
                device_id_type=pl.DeviceIdType.MESH,
            )
            rdma.start()
            a_sends.append(rdma)
        for d in range(1, N_DEV):
            src = (me - d + N_DEV) % N_DEV
            recv = pltpu.make_async_remote_copy(
                src_ref=amax_ref.at[src],
                dst_ref=amax_ref.at[src],
                send_sem=a_send_sems.at[src],
                recv_sem=a_recv_sems.at[src],
                device_id=(src,),
                device_id_type=pl.DeviceIdType.MESH,
            )
            recv.wait_recv()

        gmax = jnp.maximum(jnp.max(amax_ref[...]), 1e-20)
        scale = gmax / 448.0
        y = jnp.maximum(acc, 0.0)
        q = (y * (448.0 / gmax)).astype(jnp.float8_e4m3fn)
        out_ref[...] = q.astype(jnp.float32) * scale

        for rdma in sends:
            rdma.wait_send()
        for rdma in a_sends:
            rdma.wait_send()

    return pl.pallas_call(
        body,
        out_shape=jax.ShapeDtypeStruct((m_blk, n), jnp.float32),
        in_specs=[pl.BlockSpec(memory_space=pltpu.VMEM),
                  pl.BlockSpec(memory_space=pltpu.VMEM)],
        out_specs=pl.BlockSpec(memory_space=pltpu.VMEM),
        scratch_shapes=[
            pltpu.VMEM((N_DEV, m_blk, k_shard), jnp.bfloat16),
            pltpu.VMEM((N_DEV, 8, 128), jnp.float32),
            pltpu.SemaphoreType.DMA((N_DEV,)),
            pltpu.SemaphoreType.DMA((N_DEV,)),
            pltpu.SemaphoreType.DMA((N_DEV,)),
            pltpu.SemaphoreType.DMA((N_DEV,)),
            pltpu.SemaphoreType.DMA,
        ],
        compiler_params=pltpu.CompilerParams(
            collective_id=0,
            vmem_limit_bytes=100 * 1024 * 1024,
        ),
    )(x, w_mat)


# device time: 46940 ns/iter; 1.5311x vs baseline; 1.5311x over previous
import jax
import jax.numpy as jnp
from jax import lax
from jax.experimental import pallas as pl
from jax.experimental.pallas import tpu as pltpu

N_DEV = 16
W_SLOTS = 4


def kernel(x, w_mat):
    m_total, k_shard = x.shape
    k_total, n = w_mat.shape
    m_blk = m_total // N_DEV

    def body(x_ref, w_ref, out_ref, xb_ref, xg_ref, wbuf_ref, amax_ref,
             send_sems, recv_sems, a_send_sems, a_recv_sems, w_sems,
             local_sem):
        me = lax.axis_index("i")

        order = [me] + [(me - d) % N_DEV for d in range(1, N_DEV)]

        w_dmas = {}

        def start_w_dma(idx):
            j = order[idx]
            slot = idx % W_SLOTS
            cp = pltpu.make_async_copy(
                w_ref.at[pl.ds(j * k_shard, k_shard), :],
                wbuf_ref.at[slot],
                w_sems.at[slot],
            )
            cp.start()
            w_dmas[idx] = cp

        for idx in range(W_SLOTS):
            start_w_dma(idx)

        barrier_sem = pltpu.get_barrier_semaphore()
        for d in range(1, N_DEV):
            peer = (me + d) % N_DEV
            pl.semaphore_signal(barrier_sem, inc=1, device_id=(peer,),
                                device_id_type=pl.DeviceIdType.MESH)

        xb_ref[...] = x_ref[...].astype(jnp.bfloat16)

        pl.semaphore_wait(barrier_sem, N_DEV - 1)

        sends = []
        for d in range(1, N_DEV):
            dest = (me + d) % N_DEV
            rdma = pltpu.make_async_remote_copy(
                src_ref=xb_ref.at[pl.ds(dest * m_blk, m_blk), :],
                dst_ref=xg_ref.at[me],
                send_sem=send_sems.at[dest],
                recv_sem=recv_sems.at[me],
                device_id=(dest,),
                device_id_type=pl.DeviceIdType.MESH,
            )
            rdma.start()
            sends.append(rdma)

        cp = pltpu.make_async_copy(
            xb_ref.at[pl.ds(me * m_blk, m_blk), :], xg_ref.at[me], local_sem)
        cp.start()
        cp.wait()

        acc = None
        for idx in range(N_DEV):
            j = order[idx]
            slot = idx % W_SLOTS
            if idx >= 1:
                recv = pltpu.make_async_remote_copy(
                    src_ref=xg_ref.at[j],
                    dst_ref=xg_ref.at[j],
                    send_sem=send_sems.at[j],
                    recv_sem=recv_sems.at[j],
                    device_id=(j,),
                    device_id_type=pl.DeviceIdType.MESH,
                )
                recv.wait_recv()
            w_dmas[idx].wait()
            b = wbuf_ref[slot].astype(jnp.bfloat16)
            a = xg_ref[j]
            d_ = lax.dot_general(a, b, (((1,), (0,)), ((), ())),
                                 preferred_element_type=jnp.float32)
            acc = d_ if acc is None else acc + d_
            if idx + W_SLOTS < N_DEV:
                start_w_dma(idx + W_SLOTS)

        local_amax = jnp.maximum(jnp.max(acc), 0.0)
        amax_ref[me] = jnp.full((8, 128), local_amax, jnp.float32)
        a_sends = []
        for d in range(1, N_DEV):
            dest = (me + d) % N_DEV
            rdma = pltpu.make_async_remote_copy(
                src_ref=amax_ref.at[me],
                dst_ref=amax_ref.at[me],
                send_sem=a_send_sems.at[dest],
                recv_sem=a_recv_sems.at[me],
                device_id=(dest,),
                device_id_type=pl.DeviceIdType.MESH,
            )
            rdma.start()
            a_sends.append(rdma)
        for d in range(1, N_DEV):
            src = (me - d) % N_DEV
            recv = pltpu.make_async_remote_copy(
                src_ref=amax_ref.at[src],
                dst_ref=amax_ref.at[src],
                send_sem=a_send_sems.at[src],
                recv_sem=a_recv_sems.at[src],
                device_id=(src,),
                device_id_type=pl.DeviceIdType.MESH,
            )
            recv.wait_recv()

        gmax = jnp.maximum(jnp.max(amax_ref[...]), 1e-20)
        scale = gmax / 448.0
        y = jnp.maximum(acc, 0.0)
        q = (y * (448.0 / gmax)).astype(jnp.float8_e4m3fn)
        out_ref[...] = q.astype(jnp.float32) * scale

        for rdma in sends:
            rdma.wait_send()
        for rdma in a_sends:
            rdma.wait_send()

    return pl.pallas_call(
        body,
        out_shape=jax.ShapeDtypeStruct((m_blk, n), jnp.float32),
        in_specs=[pl.BlockSpec(memory_space=pltpu.VMEM),
                  pl.BlockSpec(memory_space=pl.ANY)],
        out_specs=pl.BlockSpec(memory_space=pltpu.VMEM),
        scratch_shapes=[
            pltpu.VMEM((m_total, k_shard), jnp.bfloat16),
            pltpu.VMEM((N_DEV, m_blk, k_shard), jnp.bfloat16),
            pltpu.VMEM((W_SLOTS, k_shard, n), jnp.float32),
            pltpu.VMEM((N_DEV, 8, 128), jnp.float32),
            pltpu.SemaphoreType.DMA((N_DEV,)),
            pltpu.SemaphoreType.DMA((N_DEV,)),
            pltpu.SemaphoreType.DMA((N_DEV,)),
            pltpu.SemaphoreType.DMA((N_DEV,)),
            pltpu.SemaphoreType.DMA((W_SLOTS,)),
            pltpu.SemaphoreType.DMA,
        ],
        compiler_params=pltpu.CompilerParams(
            collective_id=0,
            vmem_limit_bytes=100 * 1024 * 1024,
        ),
    )(x, w_mat)
